# baseline (speedup 1.0000x reference)
import functools

import jax
import jax.numpy as jnp
from jax import lax
from jax.experimental import pallas as pl
from jax.experimental.pallas import tpu as pltpu
from jax.experimental.pallas import tpu_sc as plsc

N = 10000
E = 320000
T = 3
D_IN = 128
HID = 16
HEADS = 8
D = HID * HEADS
SEM_H = 128
OUT = 64

NW = 32
EPW = E // NW
C = 32
EPWP = 10048
NCHUNK = EPWP // C
NPAD = 10240
RPT = NPAD // 16
ZR = 128
SCOLS = 144
NB = 10
RB = N // NB

_HIGH = lax.Precision.HIGHEST


def _dot(a, b):
    return jnp.dot(a, b, preferred_element_type=jnp.float32, precision=_HIGH)


def _edge_body(fs_hbm, fd_hbm, src_hbm, dst_hbm, attn_hbm, out_hbm,
               src_c0, src_c1, dst_c0, dst_c1, dsc0, dsc1,
               rows_s0, rows_s1, rows_d0, rows_d1, wbuf0, wbuf1,
               attn_v, wtmp, atr, acc,
               sem_s0, sem_s1, sem_d0, sem_d1,
               sem_is0, sem_is1, sem_id0, sem_id1, sem_w0, sem_w1):
    c = lax.axis_index("c")
    s = lax.axis_index("s")
    wid = c * 16 + s
    ebase = wid * EPWP

    src_c = [src_c0, src_c1]
    dst_c = [dst_c0, dst_c1]
    dsc = [dsc0, dsc1]
    rows_s = [rows_s0, rows_s1]
    rows_d = [rows_d0, rows_d1]
    wbuf = [wbuf0, wbuf1]
    sem_s = [sem_s0, sem_s1]
    sem_d = [sem_d0, sem_d1]
    sem_is = [sem_is0, sem_is1]
    sem_id = [sem_id0, sem_id1]
    sem_w = [sem_w0, sem_w1]

    pltpu.sync_copy(attn_hbm, attn_v)

    zvec = jnp.zeros((16,), jnp.float32)
    lanes = lax.iota(jnp.int32, 16)

    def _zero_w(i, carry):
        for k in range(SCOLS // 16):
            wbuf0[i, pl.ds(16 * k, 16)] = zvec
            wbuf1[i, pl.ds(16 * k, 16)] = zvec
        return carry

    lax.fori_loop(0, C, _zero_w, 0)

    for h in range(HEADS):
        av = attn_v[h]

        def _rot(i, carry, av=av, h=h):
            vals = jnp.take_along_axis(av, (lanes + i) & 15, axis=0)
            atr[16 * h + i] = vals
            return carry

        lax.fori_loop(0, 16, _rot, 0)

    for r in range(RPT // C):
        pltpu.sync_copy(wbuf0, acc.at[pl.ds(s * RPT + r * C, C)])
    plsc.subcore_barrier()

    def _compute(b):
        def _group(g, carry2):
            eidx = lanes + 16 * g
            for h in range(HEADS):
                def _fa(i, acc_v, h=h):
                    fv = ((lanes + i) & 15) + 16 * h
                    x = (plsc.load_gather(rows_s[b], [eidx, fv])
                         + plsc.load_gather(rows_d[b], [eidx, fv]))
                    t = jnp.maximum(x, 0.2 * x)
                    return acc_v + t * atr[16 * h + i]

                lg = lax.fori_loop(0, 16, _fa, zvec, unroll=16)
                wtmp[h] = jnp.exp(lg)

            def _fw(j, carry3):
                hvec = (lanes + j) & 7
                wvals = plsc.load_gather(wtmp, [hvec, lanes])
                plsc.store_scatter(wbuf[b], [eidx, hvec + 128], wvals)
                return carry3

            lax.fori_loop(0, HEADS, _fw, 0, unroll=2)

            wvl = [wtmp[h] for h in range(HEADS)]

            def _fb(e, carry3):
                eg = 16 * g + e
                ev = jnp.full((16,), e, jnp.int32)
                for h in range(HEADS):
                    bc = jnp.take_along_axis(wvl[h], ev, axis=0)
                    wbuf[b][eg, pl.ds(16 * h, 16)] = (
                        rows_s[b][eg, pl.ds(16 * h, 16)] * bc)
                return carry3

            lax.fori_loop(0, 16, _fb, 0, unroll=16)
            return carry2

        lax.fori_loop(0, C // 16, _group, 0)

    def _issue_idx(ci, b):
        base = ebase + lax.rem(ci, NCHUNK) * C
        pltpu.async_copy(src_hbm.at[pl.ds(base, C)], src_c[b], sem_is[b])
        pltpu.async_copy(dst_hbm.at[pl.ds(base, C)], dst_c[b], sem_id[b])

    def _wait_idx(b):
        pltpu.make_async_copy(src_hbm.at[pl.ds(0, C)], src_c[b], sem_is[b]).wait()
        pltpu.make_async_copy(dst_hbm.at[pl.ds(0, C)], dst_c[b], sem_id[b]).wait()

    def _issue_gather(b):
        pltpu.async_copy(fs_hbm.at[src_c[b]], rows_s[b], sem_s[b])
        pltpu.async_copy(fd_hbm.at[dst_c[b]], rows_d[b], sem_d[b])

    def _wait_gather(b):
        pltpu.make_async_copy(fs_hbm.at[src_c[b]], rows_s[b], sem_s[b]).wait()
        pltpu.make_async_copy(fd_hbm.at[dst_c[b]], rows_d[b], sem_d[b]).wait()

    def _wait_scat(b):
        pltpu.make_async_copy(wbuf[b], acc.at[dsc[b]], sem_w[b]).wait()

    def _step(ci, b, first):
        if not first:
            _wait_scat(b)
        _wait_gather(b)
        for k in range(C // 16):
            dsc[b][pl.ds(16 * k, 16)] = dst_c[b][pl.ds(16 * k, 16)]
        _issue_idx(ci + 2, b)
        _wait_idx(1 - b)
        _issue_gather(1 - b)
        _compute(b)
        pltpu.async_copy(wbuf[b], acc.at[dsc[b]], sem_w[b], add=True)

    _issue_idx(0, 0)
    _wait_idx(0)
    _issue_gather(0)
    _issue_idx(1, 1)
    _step(0, 0, True)
    _step(1, 1, True)

    def _pair(k, carry):
        _step(2 * k + 2, 0, False)
        _step(2 * k + 3, 1, False)
        return carry

    lax.fori_loop(0, (NCHUNK - 2) // 2, _pair, 0)

    _wait_gather(0)
    _wait_idx(1)
    _wait_scat(0)
    _wait_scat(1)
    plsc.subcore_barrier()
    pltpu.sync_copy(acc.at[pl.ds(s * RPT, RPT)],
                    out_hbm.at[c, pl.ds(s * RPT, RPT)])


_edge_call = pl.kernel(
    _edge_body,
    out_type=jax.ShapeDtypeStruct((2, NPAD, SCOLS), jnp.float32),
    mesh=plsc.VectorSubcoreMesh(core_axis_name="c", subcore_axis_name="s"),
    compiler_params=pltpu.CompilerParams(needs_layout_passes=False,
                                         use_tc_tiling_on_sc=False),
    scratch_types=[
        pltpu.VMEM((C,), jnp.int32),
        pltpu.VMEM((C,), jnp.int32),
        pltpu.VMEM((C,), jnp.int32),
        pltpu.VMEM((C,), jnp.int32),
        pltpu.VMEM((C,), jnp.int32),
        pltpu.VMEM((C,), jnp.int32),
        pltpu.VMEM((C, D), jnp.float32),
        pltpu.VMEM((C, D), jnp.float32),
        pltpu.VMEM((C, D), jnp.float32),
        pltpu.VMEM((C, D), jnp.float32),
        pltpu.VMEM((C, SCOLS), jnp.float32),
        pltpu.VMEM((C, SCOLS), jnp.float32),
        pltpu.VMEM((HEADS, HID), jnp.float32),
        pltpu.VMEM((HEADS, 16), jnp.float32),
        pltpu.VMEM((HEADS * 16, 16), jnp.float32),
        pltpu.VMEM_SHARED((NPAD, SCOLS), jnp.float32),
        pltpu.SemaphoreType.DMA,
        pltpu.SemaphoreType.DMA,
        pltpu.SemaphoreType.DMA,
        pltpu.SemaphoreType.DMA,
        pltpu.SemaphoreType.DMA,
        pltpu.SemaphoreType.DMA,
        pltpu.SemaphoreType.DMA,
        pltpu.SemaphoreType.DMA,
        pltpu.SemaphoreType.DMA,
        pltpu.SemaphoreType.DMA,
    ],
)


def _proj_body(x_ref, ws_ref, bs_ref, wd_ref, bd_ref, fs_ref, fd_ref):
    x = x_ref[...]
    fs_ref[0] = _dot(x, ws_ref[0]) + bs_ref[0]
    fd_ref[0] = _dot(x, wd_ref[0]) + bd_ref[0]


def _proj(x, Ws, bs, Wd, bd):
    bs = bs.reshape(T, 1, D)
    bd = bd.reshape(T, 1, D)
    return pl.pallas_call(
        _proj_body,
        out_shape=(jax.ShapeDtypeStruct((T, N, D), jnp.float32),
                   jax.ShapeDtypeStruct((T, N, D), jnp.float32)),
        grid=(T, NB),
        in_specs=[
            pl.BlockSpec((RB, D_IN), lambda t, i: (i, 0)),
            pl.BlockSpec((1, D_IN, D), lambda t, i: (t, 0, 0)),
            pl.BlockSpec((1, 1, D), lambda t, i: (t, 0, 0)),
            pl.BlockSpec((1, D_IN, D), lambda t, i: (t, 0, 0)),
            pl.BlockSpec((1, 1, D), lambda t, i: (t, 0, 0)),
        ],
        out_specs=(pl.BlockSpec((1, RB, D), lambda t, i: (t, i, 0)),
                   pl.BlockSpec((1, RB, D), lambda t, i: (t, i, 0))),
    )(x, Ws, bs, Wd, bd)


def _proj2_body(z_ref, beta_ref, ws_ref, bs_ref, wd_ref, bd_ref, fs_ref, fd_ref):
    x = (beta_ref[0, 0] * z_ref[:, 0, :]
         + beta_ref[0, 1] * z_ref[:, 1, :]
         + beta_ref[0, 2] * z_ref[:, 2, :])
    fs_ref[0] = _dot(x, ws_ref[0]) + bs_ref[0]
    fd_ref[0] = _dot(x, wd_ref[0]) + bd_ref[0]


def _proj2(z, beta, Ws, bs, Wd, bd):
    bs = bs.reshape(T, 1, D)
    bd = bd.reshape(T, 1, D)
    return pl.pallas_call(
        _proj2_body,
        out_shape=(jax.ShapeDtypeStruct((T, N, D), jnp.float32),
                   jax.ShapeDtypeStruct((T, N, D), jnp.float32)),
        grid=(T, NB),
        in_specs=[
            pl.BlockSpec((RB, T, D), lambda t, i: (i, 0, 0)),
            pl.BlockSpec((1, T), lambda t, i: (0, 0)),
            pl.BlockSpec((1, D, D), lambda t, i: (t, 0, 0)),
            pl.BlockSpec((1, 1, D), lambda t, i: (t, 0, 0)),
            pl.BlockSpec((1, D, D), lambda t, i: (t, 0, 0)),
            pl.BlockSpec((1, 1, D), lambda t, i: (t, 0, 0)),
        ],
        out_specs=(pl.BlockSpec((1, RB, D), lambda t, i: (t, i, 0)),
                   pl.BlockSpec((1, RB, D), lambda t, i: (t, i, 0))),
    )(z, beta, Ws, bs, Wd, bd)


def _combine_body(a0_ref, a1_ref, a2_ref, w1_ref, b1_ref, w2_ref, z_ref, wp_ref):
    rep = (lax.broadcasted_iota(jnp.int32, (HEADS, D), 1) // HID
           == lax.broadcasted_iota(jnp.int32, (HEADS, D), 0)).astype(jnp.float32)
    parts = []
    for t, a in enumerate((a0_ref, a1_ref, a2_ref)):
        acc = a[0] + a[1]
        num = acc[:, :D]
        sv = acc[:, D:D + HEADS]
        den = _dot(sv, rep) + 1e-9
        o = num / den
        zt = jnp.where(o > 0, o, jnp.exp(o) - 1.0)
        z_ref[:, t, :] = zt
        u = jnp.tanh(_dot(zt, w1_ref[...]) + b1_ref[...])
        parts.append(jnp.sum(u * w2_ref[...]))
    wp_ref[0, 0, :] = jnp.stack(parts)


def _combine(a0, a1, a2, Sw1, Sb1, Sw2row):
    return pl.pallas_call(
        _combine_body,
        out_shape=(jax.ShapeDtypeStruct((N, T, D), jnp.float32),
                   jax.ShapeDtypeStruct((NB, 1, T), jnp.float32)),
        grid=(NB,),
        in_specs=[
            pl.BlockSpec((2, RB, SCOLS), lambda i: (0, i, 0)),
            pl.BlockSpec((2, RB, SCOLS), lambda i: (0, i, 0)),
            pl.BlockSpec((2, RB, SCOLS), lambda i: (0, i, 0)),
            pl.BlockSpec((SEM_H, SEM_H), lambda i: (0, 0)),
            pl.BlockSpec((1, SEM_H), lambda i: (0, 0)),
            pl.BlockSpec((1, SEM_H), lambda i: (0, 0)),
        ],
        out_specs=(pl.BlockSpec((RB, T, D), lambda i: (i, 0, 0)),
                   pl.BlockSpec((1, 1, T), lambda i: (i, 0, 0))),
    )(a0, a1, a2, Sw1, Sb1.reshape(1, SEM_H), Sw2row)


def _final_body(z_ref, beta_ref, wf_ref, bf_ref, o_ref):
    x = (beta_ref[0, 0] * z_ref[:, 0, :]
         + beta_ref[0, 1] * z_ref[:, 1, :]
         + beta_ref[0, 2] * z_ref[:, 2, :])
    o_ref[...] = _dot(x, wf_ref[...]) + bf_ref[...]


def _final(z, beta, Wf, bf):
    return pl.pallas_call(
        _final_body,
        out_shape=jax.ShapeDtypeStruct((N, OUT), jnp.float32),
        grid=(NB,),
        in_specs=[
            pl.BlockSpec((RB, T, D), lambda i: (i, 0, 0)),
            pl.BlockSpec((1, T), lambda i: (0, 0)),
            pl.BlockSpec((D, OUT), lambda i: (0, 0)),
            pl.BlockSpec((1, OUT), lambda i: (0, 0)),
        ],
        out_specs=pl.BlockSpec((RB, OUT), lambda i: (i, 0)),
    )(z, beta, Wf, bf.reshape(1, OUT))


def _pad_edges(ei):
    pad = EPWP - EPW
    src2 = jnp.concatenate(
        [ei[0].reshape(NW, EPW),
         jnp.zeros((NW, pad), jnp.int32)], axis=1).reshape(-1)
    dst2 = jnp.concatenate(
        [ei[1].reshape(NW, EPW),
         jnp.full((NW, pad), NPAD - 1, jnp.int32)], axis=1).reshape(-1)
    return src2, dst2


def _layer(x_z, eis, proj_fn, proj_args, attn, Sw1, Sb1, Sw2):
    fs_all, fd_all = proj_fn(*proj_args)
    accs = [_edge_call(fs_all[t], fd_all[t], eis[t][0], eis[t][1], attn[t])
            for t in range(T)]
    z, wp = _combine(accs[0], accs[1], accs[2], Sw1, Sb1, Sw2.reshape(1, SEM_H))
    beta = jax.nn.softmax(wp.sum(axis=(0, 1)) / N).reshape(1, T)
    return z, beta


def kernel(h, edge_index_0, edge_index_1, edge_index_2, node_nums, Wsrc0, bsrc0, Wdst0, bdst0, attn0, Sw1_0, Sb1_0, Sw2_0, Wsrc1, bsrc1, Wdst1, bdst1, attn1, Sw1_1, Sb1_1, Sw2_1, Wf, bf):
    eis = [_pad_edges(e) for e in (edge_index_0, edge_index_1, edge_index_2)]
    z1, beta1 = _layer(h, eis, _proj, (h, Wsrc0, bsrc0, Wdst0, bdst0),
                       attn0, Sw1_0, Sb1_0, Sw2_0)
    z2, beta2 = _layer(None, eis, _proj2, (z1, beta1, Wsrc1, bsrc1, Wdst1, bdst1),
                       attn1, Sw1_1, Sb1_1, Sw2_1)
    return _final(z2, beta2, Wf, bf)


# speedup vs baseline: 1.1585x; 1.1585x over previous
import functools

import jax
import jax.numpy as jnp
from jax import lax
from jax.experimental import pallas as pl
from jax.experimental.pallas import tpu as pltpu
from jax.experimental.pallas import tpu_sc as plsc

N = 10000
E = 320000
T = 3
D_IN = 128
HID = 16
HEADS = 8
D = HID * HEADS
SEM_H = 128
OUT = 64

NW = 32
EPW = E // NW
C = 32
EPWP = 10048
NCHUNK = EPWP // C
NPAD = 10240
RPT = NPAD // 16
ZR = 128
SCOLS = 144
NB = 10
RB = N // NB

_HIGH = lax.Precision.HIGHEST


def _dot(a, b):
    return jnp.dot(a, b, preferred_element_type=jnp.float32, precision=_HIGH)


def _edge_body(fs_hbm, fd_hbm, src_hbm, dst_hbm, attn_hbm, out_hbm,
               src_c0, src_c1, dst_c0, dst_c1, dsc0, dsc1,
               rows_s0, rows_s1, rows_d0, rows_d1, wbuf0, wbuf1,
               attn_v, wtmp, atr, acc,
               sem_s0, sem_s1, sem_d0, sem_d1,
               sem_is0, sem_is1, sem_id0, sem_id1, sem_w0, sem_w1):
    c = lax.axis_index("c")
    s = lax.axis_index("s")
    wid = c * 16 + s
    ebase = wid * EPWP

    src_c = [src_c0, src_c1]
    dst_c = [dst_c0, dst_c1]
    dsc = [dsc0, dsc1]
    rows_s = [rows_s0, rows_s1]
    rows_d = [rows_d0, rows_d1]
    wbuf = [wbuf0, wbuf1]
    sem_s = [sem_s0, sem_s1]
    sem_d = [sem_d0, sem_d1]
    sem_is = [sem_is0, sem_is1]
    sem_id = [sem_id0, sem_id1]
    sem_w = [sem_w0, sem_w1]

    pltpu.sync_copy(attn_hbm, attn_v)

    zvec = jnp.zeros((16,), jnp.float32)
    lanes = lax.iota(jnp.int32, 16)

    def _zero_w(i, carry):
        for k in range(SCOLS // 16):
            wbuf0[i, pl.ds(16 * k, 16)] = zvec
            wbuf1[i, pl.ds(16 * k, 16)] = zvec
        return carry

    lax.fori_loop(0, C, _zero_w, 0)

    for h in range(HEADS):
        av = attn_v[h]

        def _rot(i, carry, av=av, h=h):
            vals = jnp.take_along_axis(av, (lanes + i) & 15, axis=0)
            atr[16 * h + i] = vals
            return carry

        lax.fori_loop(0, 16, _rot, 0)

    for r in range(RPT // C):
        pltpu.sync_copy(wbuf0, acc.at[pl.ds(s * RPT + r * C, C)])
    plsc.subcore_barrier()

    def _compute(b):
        def _group(g, carry2):
            eidx = lanes + 16 * g
            for h in range(HEADS):
                def _fa(i, acc_v, h=h):
                    fv = ((lanes + i) & 15) + 16 * h
                    x = (plsc.load_gather(rows_s[b], [eidx, fv])
                         + plsc.load_gather(rows_d[b], [eidx, fv]))
                    t = jnp.maximum(x, 0.2 * x)
                    return acc_v + t * atr[16 * h + i]

                lg = lax.fori_loop(0, 16, _fa, zvec, unroll=8)
                wtmp[h] = jnp.exp(lg)

            def _fw(j, carry3):
                hvec = (lanes + j) & 7
                wvals = plsc.load_gather(wtmp, [hvec, lanes])
                plsc.store_scatter(wbuf[b], [eidx, hvec + 128], wvals)
                return carry3

            lax.fori_loop(0, HEADS, _fw, 0, unroll=2)

            wvl = [wtmp[h] for h in range(HEADS)]

            def _fb(e, carry3):
                eg = 16 * g + e
                ev = jnp.full((16,), e, jnp.int32)
                for h in range(HEADS):
                    bc = jnp.take_along_axis(wvl[h], ev, axis=0)
                    wbuf[b][eg, pl.ds(16 * h, 16)] = (
                        rows_s[b][eg, pl.ds(16 * h, 16)] * bc)
                return carry3

            lax.fori_loop(0, 16, _fb, 0, unroll=16)
            return carry2

        lax.fori_loop(0, C // 16, _group, 0)

    def _issue_idx(ci, b):
        base = ebase + lax.rem(ci, NCHUNK) * C
        pltpu.async_copy(src_hbm.at[pl.ds(base, C)], src_c[b], sem_is[b])
        pltpu.async_copy(dst_hbm.at[pl.ds(base, C)], dst_c[b], sem_id[b])

    def _wait_idx(b):
        pltpu.make_async_copy(src_hbm.at[pl.ds(0, C)], src_c[b], sem_is[b]).wait()
        pltpu.make_async_copy(dst_hbm.at[pl.ds(0, C)], dst_c[b], sem_id[b]).wait()

    def _issue_gather(b):
        pltpu.async_copy(fs_hbm.at[src_c[b]], rows_s[b], sem_s[b])
        pltpu.async_copy(fd_hbm.at[dst_c[b]], rows_d[b], sem_d[b])

    def _wait_gather(b):
        pltpu.make_async_copy(fs_hbm.at[src_c[b]], rows_s[b], sem_s[b]).wait()
        pltpu.make_async_copy(fd_hbm.at[dst_c[b]], rows_d[b], sem_d[b]).wait()

    def _wait_scat(b):
        pltpu.make_async_copy(wbuf[b], acc.at[dsc[b]], sem_w[b]).wait()

    def _step(ci, b, first):
        if not first:
            _wait_scat(b)
        _wait_gather(b)
        for k in range(C // 16):
            dsc[b][pl.ds(16 * k, 16)] = dst_c[b][pl.ds(16 * k, 16)]
        _issue_idx(ci + 2, b)
        _wait_idx(1 - b)
        _issue_gather(1 - b)
        _compute(b)
        pltpu.async_copy(wbuf[b], acc.at[dsc[b]], sem_w[b], add=True)

    _issue_idx(0, 0)
    _wait_idx(0)
    _issue_gather(0)
    _issue_idx(1, 1)
    _step(0, 0, True)
    _step(1, 1, True)

    def _pair(k, carry):
        _step(2 * k + 2, 0, False)
        _step(2 * k + 3, 1, False)
        return carry

    lax.fori_loop(0, (NCHUNK - 2) // 2, _pair, 0)

    _wait_gather(0)
    _wait_idx(1)
    _wait_scat(0)
    _wait_scat(1)
    plsc.subcore_barrier()
    pltpu.sync_copy(acc.at[pl.ds(s * RPT, RPT)],
                    out_hbm.at[c, pl.ds(s * RPT, RPT)])


_edge_call = pl.kernel(
    _edge_body,
    out_type=jax.ShapeDtypeStruct((2, NPAD, SCOLS), jnp.float32),
    mesh=plsc.VectorSubcoreMesh(core_axis_name="c", subcore_axis_name="s"),
    compiler_params=pltpu.CompilerParams(needs_layout_passes=False,
                                         use_tc_tiling_on_sc=False),
    scratch_types=[
        pltpu.VMEM((C,), jnp.int32),
        pltpu.VMEM((C,), jnp.int32),
        pltpu.VMEM((C,), jnp.int32),
        pltpu.VMEM((C,), jnp.int32),
        pltpu.VMEM((C,), jnp.int32),
        pltpu.VMEM((C,), jnp.int32),
        pltpu.VMEM((C, D), jnp.float32),
        pltpu.VMEM((C, D), jnp.float32),
        pltpu.VMEM((C, D), jnp.float32),
        pltpu.VMEM((C, D), jnp.float32),
        pltpu.VMEM((C, SCOLS), jnp.float32),
        pltpu.VMEM((C, SCOLS), jnp.float32),
        pltpu.VMEM((HEADS, HID), jnp.float32),
        pltpu.VMEM((HEADS, 16), jnp.float32),
        pltpu.VMEM((HEADS * 16, 16), jnp.float32),
        pltpu.VMEM_SHARED((NPAD, SCOLS), jnp.float32),
        pltpu.SemaphoreType.DMA,
        pltpu.SemaphoreType.DMA,
        pltpu.SemaphoreType.DMA,
        pltpu.SemaphoreType.DMA,
        pltpu.SemaphoreType.DMA,
        pltpu.SemaphoreType.DMA,
        pltpu.SemaphoreType.DMA,
        pltpu.SemaphoreType.DMA,
        pltpu.SemaphoreType.DMA,
        pltpu.SemaphoreType.DMA,
    ],
)


def _proj_body(x_ref, ws_ref, bs_ref, wd_ref, bd_ref, fs_ref, fd_ref):
    x = x_ref[...]
    fs_ref[0] = _dot(x, ws_ref[0]) + bs_ref[0]
    fd_ref[0] = _dot(x, wd_ref[0]) + bd_ref[0]


def _proj(x, Ws, bs, Wd, bd):
    bs = bs.reshape(T, 1, D)
    bd = bd.reshape(T, 1, D)
    return pl.pallas_call(
        _proj_body,
        out_shape=(jax.ShapeDtypeStruct((T, N, D), jnp.float32),
                   jax.ShapeDtypeStruct((T, N, D), jnp.float32)),
        grid=(T, NB),
        in_specs=[
            pl.BlockSpec((RB, D_IN), lambda t, i: (i, 0)),
            pl.BlockSpec((1, D_IN, D), lambda t, i: (t, 0, 0)),
            pl.BlockSpec((1, 1, D), lambda t, i: (t, 0, 0)),
            pl.BlockSpec((1, D_IN, D), lambda t, i: (t, 0, 0)),
            pl.BlockSpec((1, 1, D), lambda t, i: (t, 0, 0)),
        ],
        out_specs=(pl.BlockSpec((1, RB, D), lambda t, i: (t, i, 0)),
                   pl.BlockSpec((1, RB, D), lambda t, i: (t, i, 0))),
    )(x, Ws, bs, Wd, bd)


def _proj2_body(z_ref, beta_ref, ws_ref, bs_ref, wd_ref, bd_ref, fs_ref, fd_ref):
    x = (beta_ref[0, 0] * z_ref[:, 0, :]
         + beta_ref[0, 1] * z_ref[:, 1, :]
         + beta_ref[0, 2] * z_ref[:, 2, :])
    fs_ref[0] = _dot(x, ws_ref[0]) + bs_ref[0]
    fd_ref[0] = _dot(x, wd_ref[0]) + bd_ref[0]


def _proj2(z, beta, Ws, bs, Wd, bd):
    bs = bs.reshape(T, 1, D)
    bd = bd.reshape(T, 1, D)
    return pl.pallas_call(
        _proj2_body,
        out_shape=(jax.ShapeDtypeStruct((T, N, D), jnp.float32),
                   jax.ShapeDtypeStruct((T, N, D), jnp.float32)),
        grid=(T, NB),
        in_specs=[
            pl.BlockSpec((RB, T, D), lambda t, i: (i, 0, 0)),
            pl.BlockSpec((1, T), lambda t, i: (0, 0)),
            pl.BlockSpec((1, D, D), lambda t, i: (t, 0, 0)),
            pl.BlockSpec((1, 1, D), lambda t, i: (t, 0, 0)),
            pl.BlockSpec((1, D, D), lambda t, i: (t, 0, 0)),
            pl.BlockSpec((1, 1, D), lambda t, i: (t, 0, 0)),
        ],
        out_specs=(pl.BlockSpec((1, RB, D), lambda t, i: (t, i, 0)),
                   pl.BlockSpec((1, RB, D), lambda t, i: (t, i, 0))),
    )(z, beta, Ws, bs, Wd, bd)


def _combine_body(a0_ref, a1_ref, a2_ref, w1_ref, b1_ref, w2_ref, z_ref, wp_ref):
    rep = (lax.broadcasted_iota(jnp.int32, (HEADS, D), 1) // HID
           == lax.broadcasted_iota(jnp.int32, (HEADS, D), 0)).astype(jnp.float32)
    parts = []
    for t, a in enumerate((a0_ref, a1_ref, a2_ref)):
        acc = a[0] + a[1]
        num = acc[:, :D]
        sv = acc[:, D:D + HEADS]
        den = _dot(sv, rep) + 1e-9
        o = num / den
        zt = jnp.where(o > 0, o, jnp.exp(o) - 1.0)
        z_ref[:, t, :] = zt
        u = jnp.tanh(_dot(zt, w1_ref[...]) + b1_ref[...])
        parts.append(jnp.sum(u * w2_ref[...]))
    wp_ref[0, 0, :] = jnp.stack(parts)


def _combine(a0, a1, a2, Sw1, Sb1, Sw2row):
    return pl.pallas_call(
        _combine_body,
        out_shape=(jax.ShapeDtypeStruct((N, T, D), jnp.float32),
                   jax.ShapeDtypeStruct((NB, 1, T), jnp.float32)),
        grid=(NB,),
        in_specs=[
            pl.BlockSpec((2, RB, SCOLS), lambda i: (0, i, 0)),
            pl.BlockSpec((2, RB, SCOLS), lambda i: (0, i, 0)),
            pl.BlockSpec((2, RB, SCOLS), lambda i: (0, i, 0)),
            pl.BlockSpec((SEM_H, SEM_H), lambda i: (0, 0)),
            pl.BlockSpec((1, SEM_H), lambda i: (0, 0)),
            pl.BlockSpec((1, SEM_H), lambda i: (0, 0)),
        ],
        out_specs=(pl.BlockSpec((RB, T, D), lambda i: (i, 0, 0)),
                   pl.BlockSpec((1, 1, T), lambda i: (i, 0, 0))),
    )(a0, a1, a2, Sw1, Sb1.reshape(1, SEM_H), Sw2row)


def _final_body(z_ref, beta_ref, wf_ref, bf_ref, o_ref):
    x = (beta_ref[0, 0] * z_ref[:, 0, :]
         + beta_ref[0, 1] * z_ref[:, 1, :]
         + beta_ref[0, 2] * z_ref[:, 2, :])
    o_ref[...] = _dot(x, wf_ref[...]) + bf_ref[...]


def _final(z, beta, Wf, bf):
    return pl.pallas_call(
        _final_body,
        out_shape=jax.ShapeDtypeStruct((N, OUT), jnp.float32),
        grid=(NB,),
        in_specs=[
            pl.BlockSpec((RB, T, D), lambda i: (i, 0, 0)),
            pl.BlockSpec((1, T), lambda i: (0, 0)),
            pl.BlockSpec((D, OUT), lambda i: (0, 0)),
            pl.BlockSpec((1, OUT), lambda i: (0, 0)),
        ],
        out_specs=pl.BlockSpec((RB, OUT), lambda i: (i, 0)),
    )(z, beta, Wf, bf.reshape(1, OUT))


def _pad_edges(ei):
    pad = EPWP - EPW
    src2 = jnp.concatenate(
        [ei[0].reshape(NW, EPW),
         jnp.zeros((NW, pad), jnp.int32)], axis=1).reshape(-1)
    dst2 = jnp.concatenate(
        [ei[1].reshape(NW, EPW),
         jnp.full((NW, pad), NPAD - 1, jnp.int32)], axis=1).reshape(-1)
    return src2, dst2


def _layer(x_z, eis, proj_fn, proj_args, attn, Sw1, Sb1, Sw2):
    fs_all, fd_all = proj_fn(*proj_args)
    accs = [_edge_call(fs_all[t], fd_all[t], eis[t][0], eis[t][1], attn[t])
            for t in range(T)]
    z, wp = _combine(accs[0], accs[1], accs[2], Sw1, Sb1, Sw2.reshape(1, SEM_H))
    beta = jax.nn.softmax(wp.sum(axis=(0, 1)) / N).reshape(1, T)
    return z, beta


def kernel(h, edge_index_0, edge_index_1, edge_index_2, node_nums, Wsrc0, bsrc0, Wdst0, bdst0, attn0, Sw1_0, Sb1_0, Sw2_0, Wsrc1, bsrc1, Wdst1, bdst1, attn1, Sw1_1, Sb1_1, Sw2_1, Wf, bf):
    eis = [_pad_edges(e) for e in (edge_index_0, edge_index_1, edge_index_2)]
    z1, beta1 = _layer(h, eis, _proj, (h, Wsrc0, bsrc0, Wdst0, bdst0),
                       attn0, Sw1_0, Sb1_0, Sw2_0)
    z2, beta2 = _layer(None, eis, _proj2, (z1, beta1, Wsrc1, bsrc1, Wdst1, bdst1),
                       attn1, Sw1_1, Sb1_1, Sw2_1)
    return _final(z2, beta2, Wf, bf)


# speedup vs baseline: 1.3056x; 1.1270x over previous
import functools

import jax
import jax.numpy as jnp
from jax import lax
from jax.experimental import pallas as pl
from jax.experimental.pallas import tpu as pltpu
from jax.experimental.pallas import tpu_sc as plsc

N = 10000
E = 320000
T = 3
D_IN = 128
HID = 16
HEADS = 8
D = HID * HEADS
SEM_H = 128
OUT = 64

NW = 32
EPW = E // NW
C = 32
EPWP = 10048
NCHUNK = EPWP // C
NPAD = 10240
RPT = NPAD // 16
ZR = 128
SCOLS = 144
NB = 10
RB = N // NB

_HIGH = lax.Precision.HIGHEST


def _dot(a, b):
    return jnp.dot(a, b, preferred_element_type=jnp.float32, precision=_HIGH)


def _edge_body(fs_hbm, fd_hbm, src_hbm, dst_hbm, attn_hbm, out_hbm,
               src_c0, src_c1, dst_c0, dst_c1, dsc0, dsc1,
               rows_s0, rows_s1, rows_d0, rows_d1, wbuf0, wbuf1,
               attn_v, wtmp, atr, acc,
               sem_s0, sem_s1, sem_d0, sem_d1,
               sem_is0, sem_is1, sem_id0, sem_id1, sem_w0, sem_w1):
    c = lax.axis_index("c")
    s = lax.axis_index("s")
    wid = c * 16 + s
    ebase = wid * EPWP

    src_c = [src_c0, src_c1]
    dst_c = [dst_c0, dst_c1]
    dsc = [dsc0, dsc1]
    rows_s = [rows_s0, rows_s1]
    rows_d = [rows_d0, rows_d1]
    wbuf = [wbuf0, wbuf1]
    sem_s = [sem_s0, sem_s1]
    sem_d = [sem_d0, sem_d1]
    sem_is = [sem_is0, sem_is1]
    sem_id = [sem_id0, sem_id1]
    sem_w = [sem_w0, sem_w1]

    pltpu.sync_copy(attn_hbm, attn_v)

    zvec = jnp.zeros((16,), jnp.float32)
    lanes = lax.iota(jnp.int32, 16)

    def _zero_w(i, carry):
        for k in range(SCOLS // 16):
            wbuf0[i, pl.ds(16 * k, 16)] = zvec
            wbuf1[i, pl.ds(16 * k, 16)] = zvec
        return carry

    lax.fori_loop(0, C, _zero_w, 0)

    for h in range(HEADS):
        av = attn_v[h]

        def _rot(i, carry, av=av, h=h):
            vals = jnp.take_along_axis(av, (lanes + i) & 15, axis=0)
            atr[16 * h + i] = vals
            return carry

        lax.fori_loop(0, 16, _rot, 0)

    for r in range(RPT // C):
        pltpu.sync_copy(wbuf0, acc.at[pl.ds(s * RPT + r * C, C)])
    plsc.subcore_barrier()

    def _compute(b):
        def _group(g, carry2):
            eidx = lanes + 16 * g
            for h in range(HEADS):
                def _fa(i, acc_v, h=h):
                    fv = ((lanes + i) & 15) + 16 * h
                    x = (plsc.load_gather(rows_s[b], [eidx, fv])
                         + plsc.load_gather(rows_d[b], [eidx, fv]))
                    t = jnp.maximum(x, 0.2 * x)
                    return acc_v + t * atr[16 * h + i]

                lg = lax.fori_loop(0, 16, _fa, zvec, unroll=8)
                wtmp[h] = jnp.exp(lg)

            def _fw(j, carry3):
                hvec = (lanes + j) & 7
                wvals = plsc.load_gather(wtmp, [hvec, lanes])
                plsc.store_scatter(wbuf[b], [eidx, hvec + 128], wvals)
                return carry3

            lax.fori_loop(0, HEADS, _fw, 0, unroll=2)

            wvl = [wtmp[h] for h in range(HEADS)]

            def _fb(e, carry3):
                eg = 16 * g + e
                ev = jnp.full((16,), e, jnp.int32)
                for h in range(HEADS):
                    bc = jnp.take_along_axis(wvl[h], ev, axis=0)
                    wbuf[b][eg, pl.ds(16 * h, 16)] = (
                        rows_s[b][eg, pl.ds(16 * h, 16)] * bc)
                return carry3

            lax.fori_loop(0, 16, _fb, 0, unroll=8)
            return carry2

        lax.fori_loop(0, C // 16, _group, 0)

    def _issue_idx(ci, b):
        base = ebase + lax.rem(ci, NCHUNK) * C
        pltpu.async_copy(src_hbm.at[pl.ds(base, C)], src_c[b], sem_is[b])
        pltpu.async_copy(dst_hbm.at[pl.ds(base, C)], dst_c[b], sem_id[b])

    def _wait_idx(b):
        pltpu.make_async_copy(src_hbm.at[pl.ds(0, C)], src_c[b], sem_is[b]).wait()
        pltpu.make_async_copy(dst_hbm.at[pl.ds(0, C)], dst_c[b], sem_id[b]).wait()

    def _issue_gather(b):
        pltpu.async_copy(fs_hbm.at[src_c[b]], rows_s[b], sem_s[b])
        pltpu.async_copy(fd_hbm.at[dst_c[b]], rows_d[b], sem_d[b])

    def _wait_gather(b):
        pltpu.make_async_copy(fs_hbm.at[src_c[b]], rows_s[b], sem_s[b]).wait()
        pltpu.make_async_copy(fd_hbm.at[dst_c[b]], rows_d[b], sem_d[b]).wait()

    def _wait_scat(b):
        pltpu.make_async_copy(wbuf[b], acc.at[dsc[b]], sem_w[b]).wait()

    def _step(ci, b, first):
        if not first:
            _wait_scat(b)
        _wait_gather(b)
        for k in range(C // 16):
            dsc[b][pl.ds(16 * k, 16)] = dst_c[b][pl.ds(16 * k, 16)]
        _issue_idx(ci + 2, b)
        _wait_idx(1 - b)
        _issue_gather(1 - b)
        _compute(b)
        pltpu.async_copy(wbuf[b], acc.at[dsc[b]], sem_w[b], add=True)

    _issue_idx(0, 0)
    _wait_idx(0)
    _issue_gather(0)
    _issue_idx(1, 1)
    _step(0, 0, True)
    _step(1, 1, True)

    def _pair(k, carry):
        _step(2 * k + 2, 0, False)
        _step(2 * k + 3, 1, False)
        return carry

    lax.fori_loop(0, (NCHUNK - 2) // 2, _pair, 0)

    _wait_gather(0)
    _wait_idx(1)
    _wait_scat(0)
    _wait_scat(1)
    plsc.subcore_barrier()
    pltpu.sync_copy(acc.at[pl.ds(s * RPT, RPT)],
                    out_hbm.at[c, pl.ds(s * RPT, RPT)])


_edge_call = pl.kernel(
    _edge_body,
    out_type=jax.ShapeDtypeStruct((2, NPAD, SCOLS), jnp.float32),
    mesh=plsc.VectorSubcoreMesh(core_axis_name="c", subcore_axis_name="s"),
    compiler_params=pltpu.CompilerParams(needs_layout_passes=False,
                                         use_tc_tiling_on_sc=False),
    scratch_types=[
        pltpu.VMEM((C,), jnp.int32),
        pltpu.VMEM((C,), jnp.int32),
        pltpu.VMEM((C,), jnp.int32),
        pltpu.VMEM((C,), jnp.int32),
        pltpu.VMEM((C,), jnp.int32),
        pltpu.VMEM((C,), jnp.int32),
        pltpu.VMEM((C, D), jnp.float32),
        pltpu.VMEM((C, D), jnp.float32),
        pltpu.VMEM((C, D), jnp.float32),
        pltpu.VMEM((C, D), jnp.float32),
        pltpu.VMEM((C, SCOLS), jnp.float32),
        pltpu.VMEM((C, SCOLS), jnp.float32),
        pltpu.VMEM((HEADS, HID), jnp.float32),
        pltpu.VMEM((HEADS, 16), jnp.float32),
        pltpu.VMEM((HEADS * 16, 16), jnp.float32),
        pltpu.VMEM_SHARED((NPAD, SCOLS), jnp.float32),
        pltpu.SemaphoreType.DMA,
        pltpu.SemaphoreType.DMA,
        pltpu.SemaphoreType.DMA,
        pltpu.SemaphoreType.DMA,
        pltpu.SemaphoreType.DMA,
        pltpu.SemaphoreType.DMA,
        pltpu.SemaphoreType.DMA,
        pltpu.SemaphoreType.DMA,
        pltpu.SemaphoreType.DMA,
        pltpu.SemaphoreType.DMA,
    ],
)


def _proj_body(x_ref, ws_ref, bs_ref, wd_ref, bd_ref, fs_ref, fd_ref):
    x = x_ref[...]
    fs_ref[0] = _dot(x, ws_ref[0]) + bs_ref[0]
    fd_ref[0] = _dot(x, wd_ref[0]) + bd_ref[0]


def _proj(x, Ws, bs, Wd, bd):
    bs = bs.reshape(T, 1, D)
    bd = bd.reshape(T, 1, D)
    return pl.pallas_call(
        _proj_body,
        out_shape=(jax.ShapeDtypeStruct((T, N, D), jnp.float32),
                   jax.ShapeDtypeStruct((T, N, D), jnp.float32)),
        grid=(T, NB),
        in_specs=[
            pl.BlockSpec((RB, D_IN), lambda t, i: (i, 0)),
            pl.BlockSpec((1, D_IN, D), lambda t, i: (t, 0, 0)),
            pl.BlockSpec((1, 1, D), lambda t, i: (t, 0, 0)),
            pl.BlockSpec((1, D_IN, D), lambda t, i: (t, 0, 0)),
            pl.BlockSpec((1, 1, D), lambda t, i: (t, 0, 0)),
        ],
        out_specs=(pl.BlockSpec((1, RB, D), lambda t, i: (t, i, 0)),
                   pl.BlockSpec((1, RB, D), lambda t, i: (t, i, 0))),
    )(x, Ws, bs, Wd, bd)


def _proj2_body(z_ref, beta_ref, ws_ref, bs_ref, wd_ref, bd_ref, fs_ref, fd_ref):
    x = (beta_ref[0, 0] * z_ref[:, 0, :]
         + beta_ref[0, 1] * z_ref[:, 1, :]
         + beta_ref[0, 2] * z_ref[:, 2, :])
    fs_ref[0] = _dot(x, ws_ref[0]) + bs_ref[0]
    fd_ref[0] = _dot(x, wd_ref[0]) + bd_ref[0]


def _proj2(z, beta, Ws, bs, Wd, bd):
    bs = bs.reshape(T, 1, D)
    bd = bd.reshape(T, 1, D)
    return pl.pallas_call(
        _proj2_body,
        out_shape=(jax.ShapeDtypeStruct((T, N, D), jnp.float32),
                   jax.ShapeDtypeStruct((T, N, D), jnp.float32)),
        grid=(T, NB),
        in_specs=[
            pl.BlockSpec((RB, T, D), lambda t, i: (i, 0, 0)),
            pl.BlockSpec((1, T), lambda t, i: (0, 0)),
            pl.BlockSpec((1, D, D), lambda t, i: (t, 0, 0)),
            pl.BlockSpec((1, 1, D), lambda t, i: (t, 0, 0)),
            pl.BlockSpec((1, D, D), lambda t, i: (t, 0, 0)),
            pl.BlockSpec((1, 1, D), lambda t, i: (t, 0, 0)),
        ],
        out_specs=(pl.BlockSpec((1, RB, D), lambda t, i: (t, i, 0)),
                   pl.BlockSpec((1, RB, D), lambda t, i: (t, i, 0))),
    )(z, beta, Ws, bs, Wd, bd)


def _combine_body(a0_ref, a1_ref, a2_ref, w1_ref, b1_ref, w2_ref, z_ref, wp_ref):
    rep = (lax.broadcasted_iota(jnp.int32, (HEADS, D), 1) // HID
           == lax.broadcasted_iota(jnp.int32, (HEADS, D), 0)).astype(jnp.float32)
    parts = []
    for t, a in enumerate((a0_ref, a1_ref, a2_ref)):
        acc = a[0] + a[1]
        num = acc[:, :D]
        sv = acc[:, D:D + HEADS]
        den = _dot(sv, rep) + 1e-9
        o = num / den
        zt = jnp.where(o > 0, o, jnp.exp(o) - 1.0)
        z_ref[:, t, :] = zt
        u = jnp.tanh(_dot(zt, w1_ref[...]) + b1_ref[...])
        parts.append(jnp.sum(u * w2_ref[...]))
    wp_ref[0, 0, :] = jnp.stack(parts)


def _combine(a0, a1, a2, Sw1, Sb1, Sw2row):
    return pl.pallas_call(
        _combine_body,
        out_shape=(jax.ShapeDtypeStruct((N, T, D), jnp.float32),
                   jax.ShapeDtypeStruct((NB, 1, T), jnp.float32)),
        grid=(NB,),
        in_specs=[
            pl.BlockSpec((2, RB, SCOLS), lambda i: (0, i, 0)),
            pl.BlockSpec((2, RB, SCOLS), lambda i: (0, i, 0)),
            pl.BlockSpec((2, RB, SCOLS), lambda i: (0, i, 0)),
            pl.BlockSpec((SEM_H, SEM_H), lambda i: (0, 0)),
            pl.BlockSpec((1, SEM_H), lambda i: (0, 0)),
            pl.BlockSpec((1, SEM_H), lambda i: (0, 0)),
        ],
        out_specs=(pl.BlockSpec((RB, T, D), lambda i: (i, 0, 0)),
                   pl.BlockSpec((1, 1, T), lambda i: (i, 0, 0))),
    )(a0, a1, a2, Sw1, Sb1.reshape(1, SEM_H), Sw2row)


def _final_body(z_ref, beta_ref, wf_ref, bf_ref, o_ref):
    x = (beta_ref[0, 0] * z_ref[:, 0, :]
         + beta_ref[0, 1] * z_ref[:, 1, :]
         + beta_ref[0, 2] * z_ref[:, 2, :])
    o_ref[...] = _dot(x, wf_ref[...]) + bf_ref[...]


def _final(z, beta, Wf, bf):
    return pl.pallas_call(
        _final_body,
        out_shape=jax.ShapeDtypeStruct((N, OUT), jnp.float32),
        grid=(NB,),
        in_specs=[
            pl.BlockSpec((RB, T, D), lambda i: (i, 0, 0)),
            pl.BlockSpec((1, T), lambda i: (0, 0)),
            pl.BlockSpec((D, OUT), lambda i: (0, 0)),
            pl.BlockSpec((1, OUT), lambda i: (0, 0)),
        ],
        out_specs=pl.BlockSpec((RB, OUT), lambda i: (i, 0)),
    )(z, beta, Wf, bf.reshape(1, OUT))


def _pad_edges(ei):
    pad = EPWP - EPW
    src2 = jnp.concatenate(
        [ei[0].reshape(NW, EPW),
         jnp.zeros((NW, pad), jnp.int32)], axis=1).reshape(-1)
    dst2 = jnp.concatenate(
        [ei[1].reshape(NW, EPW),
         jnp.full((NW, pad), NPAD - 1, jnp.int32)], axis=1).reshape(-1)
    return src2, dst2


def _layer(x_z, eis, proj_fn, proj_args, attn, Sw1, Sb1, Sw2):
    fs_all, fd_all = proj_fn(*proj_args)
    accs = [_edge_call(fs_all[t], fd_all[t], eis[t][0], eis[t][1], attn[t])
            for t in range(T)]
    z, wp = _combine(accs[0], accs[1], accs[2], Sw1, Sb1, Sw2.reshape(1, SEM_H))
    beta = jax.nn.softmax(wp.sum(axis=(0, 1)) / N).reshape(1, T)
    return z, beta


def kernel(h, edge_index_0, edge_index_1, edge_index_2, node_nums, Wsrc0, bsrc0, Wdst0, bdst0, attn0, Sw1_0, Sb1_0, Sw2_0, Wsrc1, bsrc1, Wdst1, bdst1, attn1, Sw1_1, Sb1_1, Sw2_1, Wf, bf):
    eis = [_pad_edges(e) for e in (edge_index_0, edge_index_1, edge_index_2)]
    z1, beta1 = _layer(h, eis, _proj, (h, Wsrc0, bsrc0, Wdst0, bdst0),
                       attn0, Sw1_0, Sb1_0, Sw2_0)
    z2, beta2 = _layer(None, eis, _proj2, (z1, beta1, Wsrc1, bsrc1, Wdst1, bdst1),
                       attn1, Sw1_1, Sb1_1, Sw2_1)
    return _final(z2, beta2, Wf, bf)
